# baseline (speedup 1.0000x reference)
import functools
import math

import jax
import jax.numpy as jnp
import numpy as np
from jax import lax
from jax.experimental import pallas as pl
from jax.experimental.pallas import tpu as pltpu
from jax.experimental.pallas import tpu_sc as plsc

D_MODEL = 64
SEQ = 200
BATCH = 4096
GCH = 40
NG = SEQ // GCH

NUM_CORES = 2
NUM_SUBCORES = 16
NW = NUM_CORES * NUM_SUBCORES
SEQ_PER_W = BATCH // NW
NBUF = 4
ROUNDS = SEQ_PER_W // NBUF


def _pe_table() -> jnp.ndarray:
    pe = np.zeros((SEQ, D_MODEL), dtype=np.float32)
    pos = np.arange(0, SEQ, dtype=np.float32)[:, None]
    k = np.exp(-math.log(10000.0) * np.arange(0, D_MODEL, 2, dtype=np.float32) / D_MODEL)
    pe[:, 0::2] = np.sin(pos * k)
    pe[:, 1::2] = np.cos(pos * k)
    return jnp.asarray(pe)


_MESH = plsc.VectorSubcoreMesh(core_axis_name="c", subcore_axis_name="s")


@functools.partial(
    pl.kernel,
    out_type=jax.ShapeDtypeStruct((BATCH, SEQ, D_MODEL), jnp.float32),
    mesh=_MESH,
    compiler_params=pltpu.CompilerParams(use_tc_tiling_on_sc=False),
    scratch_types=[
        pltpu.VMEM((SEQ_PER_W, SEQ), jnp.int32),
        pltpu.VMEM((NBUF, SEQ, D_MODEL), jnp.float32),
        [pltpu.SemaphoreType.DMA] * NBUF,
        [pltpu.SemaphoreType.DMA] * NBUF,
        [pltpu.SemaphoreType.DMA] * NBUF,
    ],
)
def _sc_embed(table_hbm, x_hbm, pe_hbm, out_hbm, idx_v, bufs, psems, gsems, osems):
    wid = lax.axis_index("s") * NUM_CORES + lax.axis_index("c")
    seq_base = wid * SEQ_PER_W
    pltpu.sync_copy(x_hbm.at[pl.ds(seq_base, SEQ_PER_W)], idx_v)

    def one_round(g, wait_out):
        for k in range(NBUF):
            if wait_out:
                pltpu.make_async_copy(bufs.at[k], out_hbm.at[seq_base], osems[k]).wait()
            pltpu.async_copy(pe_hbm, bufs.at[k], psems[k])
        for k in range(NBUF):
            r = g * NBUF + k
            pltpu.make_async_copy(pe_hbm, bufs.at[k], psems[k]).wait()
            for h in range(NG):
                pltpu.async_copy(
                    table_hbm.at[idx_v.at[r, pl.ds(h * GCH, GCH)]],
                    bufs.at[k, pl.ds(h * GCH, GCH)],
                    gsems[k], add=True)
        for k in range(NBUF):
            r = g * NBUF + k
            for h in range(NG):
                pltpu.make_async_copy(
                    table_hbm.at[idx_v.at[r, pl.ds(h * GCH, GCH)]],
                    bufs.at[k, pl.ds(h * GCH, GCH)],
                    gsems[k]).wait()
            pltpu.async_copy(bufs.at[k], out_hbm.at[seq_base + r], osems[k])

    one_round(0, wait_out=False)

    def body(g, carry):
        one_round(g, wait_out=True)
        return carry

    lax.fori_loop(1, ROUNDS, body, 0)

    for k in range(NBUF):
        pltpu.make_async_copy(bufs.at[k], out_hbm.at[seq_base], osems[k]).wait()


def kernel(x, embedding_weight):
    return _sc_embed(embedding_weight, x.astype(jnp.int32), _pe_table())


# speedup vs baseline: 1.0016x; 1.0016x over previous
import functools
import math

import jax
import jax.numpy as jnp
import numpy as np
from jax import lax
from jax.experimental import pallas as pl
from jax.experimental.pallas import tpu as pltpu
from jax.experimental.pallas import tpu_sc as plsc

D_MODEL = 64
SEQ = 200
BATCH = 4096
VOCAB = 1000000

NUM_CORES = 2
NUM_SUBCORES = 16
NW = NUM_CORES * NUM_SUBCORES
ROWS = BATCH * SEQ
ROWS_PER_W = ROWS // NW
CHUNK = 2 * SEQ
CHUNKS_PER_W = ROWS_PER_W // CHUNK
GSIZES = (128, 128, 128, 16)
NBUF = 2
ROUNDS = CHUNKS_PER_W // NBUF


def _pe2_table() -> jnp.ndarray:
    pe = np.zeros((SEQ, D_MODEL), dtype=np.float32)
    pos = np.arange(0, SEQ, dtype=np.float32)[:, None]
    k = np.exp(-math.log(10000.0) * np.arange(0, D_MODEL, 2, dtype=np.float32) / D_MODEL)
    pe[:, 0::2] = np.sin(pos * k)
    pe[:, 1::2] = np.cos(pos * k)
    return jnp.asarray(np.concatenate([pe, pe], axis=0))


_MESH = plsc.VectorSubcoreMesh(core_axis_name="c", subcore_axis_name="s")


@functools.partial(
    pl.kernel,
    out_type=jax.ShapeDtypeStruct((ROWS, D_MODEL), jnp.float32),
    mesh=_MESH,
    compiler_params=pltpu.CompilerParams(use_tc_tiling_on_sc=False),
    scratch_types=[
        pltpu.VMEM((ROWS_PER_W,), jnp.int32),
        pltpu.VMEM((NBUF, CHUNK, D_MODEL), jnp.float32),
        [pltpu.SemaphoreType.DMA] * NBUF,
        [pltpu.SemaphoreType.DMA] * NBUF,
        [pltpu.SemaphoreType.DMA] * NBUF,
    ],
)
def _sc_embed(table_hbm, xf_hbm, pe2_hbm, out_hbm, idx_v, bufs, psems, gsems, osems):
    wid = lax.axis_index("s") * NUM_CORES + lax.axis_index("c")
    row_base = wid * ROWS_PER_W
    pltpu.sync_copy(xf_hbm.at[pl.ds(row_base, ROWS_PER_W)], idx_v)

    def one_round(g, wait_out):
        for k in range(NBUF):
            if wait_out:
                pltpu.make_async_copy(bufs.at[k], out_hbm.at[pl.ds(0, CHUNK)],
                                      osems[k]).wait()
            pltpu.async_copy(pe2_hbm, bufs.at[k], psems[k])
        for k in range(NBUF):
            c = (g * NBUF + k) * CHUNK
            pltpu.make_async_copy(pe2_hbm, bufs.at[k], psems[k]).wait()
            off = 0
            for gs in GSIZES:
                pltpu.async_copy(
                    table_hbm.at[idx_v.at[pl.ds(c + off, gs)]],
                    bufs.at[k, pl.ds(off, gs)],
                    gsems[k], add=True)
                off += gs
        for k in range(NBUF):
            c = (g * NBUF + k) * CHUNK
            off = 0
            for gs in GSIZES:
                pltpu.make_async_copy(
                    table_hbm.at[idx_v.at[pl.ds(c + off, gs)]],
                    bufs.at[k, pl.ds(off, gs)],
                    gsems[k]).wait()
                off += gs
            pltpu.async_copy(bufs.at[k], out_hbm.at[pl.ds(row_base + c, CHUNK)],
                             osems[k])

    one_round(0, wait_out=False)

    def body(g, carry):
        one_round(g, wait_out=True)
        return carry

    lax.fori_loop(1, ROUNDS, body, 0)

    for k in range(NBUF):
        pltpu.make_async_copy(bufs.at[k], out_hbm.at[pl.ds(0, CHUNK)], osems[k]).wait()


def kernel(x, embedding_weight):
    xf = jnp.clip(x.astype(jnp.int32), 0, VOCAB - 1).reshape(ROWS)
    out = _sc_embed(embedding_weight, xf, _pe2_table())
    return out.reshape(BATCH, SEQ, D_MODEL)


# speedup vs baseline: 1.2511x; 1.2491x over previous
import functools
import math

import jax
import jax.numpy as jnp
import numpy as np
from jax import lax
from jax.experimental import pallas as pl
from jax.experimental.pallas import tpu as pltpu
from jax.experimental.pallas import tpu_sc as plsc

D_MODEL = 64
SEQ = 200
BATCH = 4096
VOCAB = 1000000

NUM_CORES = 2
NUM_SUBCORES = 16
NW = NUM_CORES * NUM_SUBCORES
ROWS = BATCH * SEQ
ROWS_PER_W = ROWS // NW
CHUNK = 2 * SEQ
CHUNKS_PER_W = ROWS_PER_W // CHUNK
NCHUNKS = ROWS // CHUNK
GSIZES = (128, 128, 128, 16)
NBUF = 2
ROUNDS = CHUNKS_PER_W // NBUF
LANES = 16


def _pe2_table() -> jnp.ndarray:
    pe = np.zeros((SEQ, D_MODEL), dtype=np.float32)
    pos = np.arange(0, SEQ, dtype=np.float32)[:, None]
    k = np.exp(-math.log(10000.0) * np.arange(0, D_MODEL, 2, dtype=np.float32) / D_MODEL)
    pe[:, 0::2] = np.sin(pos * k)
    pe[:, 1::2] = np.cos(pos * k)
    return jnp.asarray(np.concatenate([pe, pe], axis=0))


_MESH = plsc.VectorSubcoreMesh(core_axis_name="c", subcore_axis_name="s")


@functools.partial(
    pl.kernel,
    out_type=jax.ShapeDtypeStruct((NCHUNKS, CHUNK, D_MODEL), jnp.float32),
    mesh=_MESH,
    compiler_params=pltpu.CompilerParams(use_tc_tiling_on_sc=False),
    scratch_types=[
        pltpu.VMEM((ROWS_PER_W,), jnp.int32),
        pltpu.VMEM((CHUNK, D_MODEL), jnp.float32),
        pltpu.VMEM((NBUF, CHUNK, D_MODEL), jnp.float32),
        [pltpu.SemaphoreType.DMA] * NBUF,
        [pltpu.SemaphoreType.DMA] * NBUF,
    ],
)
def _sc_embed(table_hbm, xf_hbm, pe2_hbm, out_hbm, idx_v, pe_v, bufs, gsems, osems):
    wid = lax.axis_index("s") * NUM_CORES + lax.axis_index("c")
    row_base = wid * ROWS_PER_W
    chunk_base = wid * CHUNKS_PER_W
    pltpu.sync_copy(xf_hbm.at[pl.ds(row_base, ROWS_PER_W)], idx_v)
    pltpu.sync_copy(pe2_hbm, pe_v)

    def pe_fill(k):
        def row(r, carry):
            for j in range(D_MODEL // LANES):
                bufs[k, r, pl.ds(j * LANES, LANES)] = pe_v[r, pl.ds(j * LANES, LANES)]
            return carry
        lax.fori_loop(0, CHUNK, row, 0)

    def one_round(g, wait_out):
        for k in range(NBUF):
            if wait_out:
                pltpu.make_async_copy(bufs.at[k], out_hbm.at[chunk_base],
                                      osems[k]).wait()
            pe_fill(k)
            c = (g * NBUF + k) * CHUNK
            off = 0
            for gs in GSIZES:
                pltpu.async_copy(
                    table_hbm.at[idx_v.at[pl.ds(c + off, gs)]],
                    bufs.at[k, pl.ds(off, gs)],
                    gsems[k], add=True)
                off += gs
        for k in range(NBUF):
            c = (g * NBUF + k) * CHUNK
            off = 0
            for gs in GSIZES:
                pltpu.make_async_copy(
                    table_hbm.at[idx_v.at[pl.ds(c + off, gs)]],
                    bufs.at[k, pl.ds(off, gs)],
                    gsems[k]).wait()
                off += gs
            pltpu.async_copy(bufs.at[k], out_hbm.at[chunk_base + g * NBUF + k],
                             osems[k])

    one_round(0, wait_out=False)

    def body(g, carry):
        one_round(g, wait_out=True)
        return carry

    lax.fori_loop(1, ROUNDS, body, 0)

    for k in range(NBUF):
        pltpu.make_async_copy(bufs.at[k], out_hbm.at[chunk_base], osems[k]).wait()


def kernel(x, embedding_weight):
    xf = jnp.clip(x.astype(jnp.int32), 0, VOCAB - 1).reshape(ROWS)
    out = _sc_embed(embedding_weight, xf, _pe2_table())
    return out.reshape(BATCH, SEQ, D_MODEL)
